# Initial kernel scaffold; baseline (speedup 1.0000x reference)
import jax
import jax.numpy as jnp
from jax.experimental import pallas as pl


def kernel(x, edge_index, W1, b1, W2, b2, W3, b3, W4, b4, W5, b5, Wp, bp):
    raise NotImplementedError("write your pallas kernel here")


# speedup vs baseline: 4.1671x; 4.1671x over previous
import functools

import jax
import jax.numpy as jnp
from jax import lax
from jax.experimental import pallas as pl
from jax.experimental.pallas import tpu as pltpu
from jax.experimental.pallas import tpu_sc as plsc

N = 10000
E = 320000
D = 128
NC = 2
NS = 16
NW = NC * NS
EPW = E // NW
C = 80
NCHUNK = EPW // C
NP = 10240
RPT = NP // NS
ZR = 128
DEGW = 128

def _fill_vmem(ref, rows, width, value):
    def body(i, _):
        for j in range(width // 16):
            ref[i, pl.ds(16 * j, 16)] = jnp.full((16,), value, jnp.float32)
        return 0
    lax.fori_loop(0, rows, body, 0, unroll=False)


@functools.cache
def _sc_kernels():
    mesh = plsc.VectorSubcoreMesh(core_axis_name="c", subcore_axis_name="s",
                                  num_cores=NC, num_subcores=NS)

    @functools.partial(
        pl.kernel,
        mesh=mesh,
        out_type=jax.ShapeDtypeStruct((NC, NP, DEGW), jnp.float32),
        scratch_types=[
            pltpu.VMEM((C,), jnp.int32),
            pltpu.VMEM((C, DEGW), jnp.float32),
            pltpu.VMEM((ZR, DEGW), jnp.float32),
            pltpu.VMEM_SHARED((NP, DEGW), jnp.float32),
        ],
    )
    def deg_kernel(dst_hbm, out_hbm, idx_v, ones_v, zbuf_v, acc_sh):
        cid = lax.axis_index("c")
        sid = lax.axis_index("s")
        wid = cid * NS + sid

        _fill_vmem(ones_v, C, DEGW, 1.0)
        _fill_vmem(zbuf_v, ZR, DEGW, 0.0)
        for k in range(RPT // ZR):
            pltpu.sync_copy(zbuf_v, acc_sh.at[pl.ds(sid * RPT + k * ZR, ZR)])
        plsc.subcore_barrier()

        base = wid * EPW

        def body(i, _):
            pltpu.sync_copy(dst_hbm.at[pl.ds(base + i * C, C)], idx_v)
            pltpu.sync_copy(ones_v, acc_sh.at[idx_v], add=True)
            return 0
        lax.fori_loop(0, NCHUNK, body, 0, unroll=False)

        plsc.subcore_barrier()
        pltpu.sync_copy(acc_sh.at[pl.ds(sid * RPT, RPT)],
                        out_hbm.at[cid, pl.ds(sid * RPT, RPT)])

    @functools.partial(
        pl.kernel,
        mesh=mesh,
        out_type=jax.ShapeDtypeStruct((NC, NP, D), jnp.float32),
        scratch_types=[
            pltpu.VMEM((C,), jnp.int32),
            pltpu.VMEM((C,), jnp.int32),
            pltpu.VMEM((C, D), jnp.float32),
            pltpu.VMEM((ZR, D), jnp.float32),
            pltpu.VMEM_SHARED((NP, D), jnp.float32),
            pltpu.SemaphoreType.DMA,
        ],
    )
    def spmm_kernel(g_hbm, src_hbm, dst_hbm, out_hbm,
                    src_v, dst_v, rows_v, zbuf_v, y_sh, sem):
        cid = lax.axis_index("c")
        sid = lax.axis_index("s")
        wid = cid * NS + sid

        _fill_vmem(zbuf_v, ZR, D, 0.0)
        for k in range(RPT // ZR):
            pltpu.sync_copy(zbuf_v, y_sh.at[pl.ds(sid * RPT + k * ZR, ZR)])
        plsc.subcore_barrier()

        base = wid * EPW

        def body(i, _):
            off = base + i * C
            pltpu.sync_copy(src_hbm.at[pl.ds(off, C)], src_v)
            pltpu.sync_copy(dst_hbm.at[pl.ds(off, C)], dst_v)
            pltpu.async_copy(g_hbm.at[src_v], rows_v, sem).wait()
            pltpu.sync_copy(rows_v, y_sh.at[dst_v], add=True)
            return 0
        lax.fori_loop(0, NCHUNK, body, 0, unroll=False)

        plsc.subcore_barrier()
        pltpu.sync_copy(y_sh.at[pl.ds(sid * RPT, RPT)],
                        out_hbm.at[cid, pl.ds(sid * RPT, RPT)])

    return deg_kernel, spmm_kernel


R = 400
GRID = N // R
_F32MAX = 3.4028234663852886e38


def _prep_body(deg2_ref, x_ref, normb_ref, normb2_ref, g1_ref, h0_ref):
    deg = deg2_ref[0, :, 0:1] + deg2_ref[1, :, 0:1]
    nrm = lax.rsqrt(jnp.maximum(deg, 1.0))
    nb = jnp.broadcast_to(nrm, (R, D))
    xb = x_ref[...]
    h0 = jnp.where(jnp.isnan(xb), 0.0, xb)
    h0 = jnp.clip(h0, -_F32MAX, _F32MAX)
    normb_ref[...] = nb
    normb2_ref[...] = nb * nb
    g1_ref[...] = nb * h0
    h0_ref[...] = h0


_prep_call = pl.pallas_call(
    _prep_body,
    grid=(GRID,),
    in_specs=[
        pl.BlockSpec((NC, R, DEGW), lambda i: (0, i, 0)),
        pl.BlockSpec((R, D), lambda i: (i, 0)),
    ],
    out_specs=[pl.BlockSpec((R, D), lambda i: (i, 0))] * 4,
    out_shape=[jax.ShapeDtypeStruct((N, D), jnp.float32)] * 4,
)


def _scale_body(ypair_ref, nb2_ref, ysum_ref, g2_ref):
    y = ypair_ref[0] + ypair_ref[1]
    ysum_ref[...] = y
    g2_ref[...] = nb2_ref[...] * y


_scale_call = pl.pallas_call(
    _scale_body,
    grid=(GRID,),
    in_specs=[
        pl.BlockSpec((NC, R, D), lambda i: (0, i, 0)),
        pl.BlockSpec((R, D), lambda i: (i, 0)),
    ],
    out_specs=[pl.BlockSpec((R, D), lambda i: (i, 0))] * 2,
    out_shape=[jax.ShapeDtypeStruct((N, D), jnp.float32)] * 2,
)


def _layer_body(h_ref, y1_ref, y2p_ref, nb_ref, w_ref, b_ref, hn_ref, gn_ref):
    y2 = y2p_ref[0] + y2p_ref[1]
    dot = functools.partial(jnp.dot, preferred_element_type=jnp.float32,
                            precision=lax.Precision.HIGHEST)
    part = dot(y1_ref[...], w_ref[1]) + dot(y2, w_ref[2])
    out = dot(h_ref[...], w_ref[0]) + nb_ref[...] * part + b_ref[...]
    hn = jnp.maximum(out, jnp.float32(0.0))
    hn_ref[...] = hn
    gn_ref[...] = nb_ref[...] * hn


_layer_call = pl.pallas_call(
    _layer_body,
    grid=(GRID,),
    in_specs=[
        pl.BlockSpec((R, D), lambda i: (i, 0)),
        pl.BlockSpec((R, D), lambda i: (i, 0)),
        pl.BlockSpec((NC, R, D), lambda i: (0, i, 0)),
        pl.BlockSpec((R, D), lambda i: (i, 0)),
        pl.BlockSpec((3, D, D), lambda i: (0, 0, 0)),
        pl.BlockSpec((1, D), lambda i: (0, 0)),
    ],
    out_specs=[pl.BlockSpec((R, D), lambda i: (i, 0))] * 2,
    out_shape=[jax.ShapeDtypeStruct((N, D), jnp.float32)] * 2,
)


def _final_body(h_ref, wpt_ref, bp_ref, out_ref, acc_ref):
    i = pl.program_id(0)

    @pl.when(i == 0)
    def _():
        acc_ref[...] = jnp.zeros_like(acc_ref)
        out_ref[...] = jnp.zeros((1, 1), jnp.float32)

    acc_ref[...] += h_ref[...].reshape(R // 8, 8, D).sum(axis=0)

    @pl.when(i == GRID - 1)
    def _():
        tot = acc_ref[...].sum(axis=0, keepdims=True)
        val = jnp.sum(tot * wpt_ref[...]) / N + bp_ref[0, 0]
        out_ref[...] = val.reshape(1, 1)


_final_call = pl.pallas_call(
    _final_body,
    grid=(GRID,),
    in_specs=[
        pl.BlockSpec((R, D), lambda i: (i, 0)),
        pl.BlockSpec((1, D), lambda i: (0, 0)),
        pl.BlockSpec((1, 1), lambda i: (0, 0)),
    ],
    out_specs=pl.BlockSpec((1, 1), lambda i: (0, 0)),
    out_shape=jax.ShapeDtypeStruct((1, 1), jnp.float32),
    scratch_shapes=[pltpu.VMEM((8, D), jnp.float32)],
)


def kernel(x, edge_index, W1, b1, W2, b2, W3, b3, W4, b4, W5, b5, Wp, bp):
    src = edge_index[0]
    dst = edge_index[1]
    deg_kernel, spmm_kernel = _sc_kernels()

    deg2 = deg_kernel(dst)
    normb, normb2, g, h = _prep_call(deg2, x)

    for W, b in ((W1, b1), (W2, b2), (W3, b3), (W4, b4), (W5, b5)):
        y1p = spmm_kernel(g, src, dst)
        y1, g2 = _scale_call(y1p, normb2)
        y2p = spmm_kernel(g2, src, dst)
        h, g = _layer_call(h, y1, y2p, normb, W.reshape(3, D, D),
                           b.reshape(1, D))

    return _final_call(h, Wp.reshape(1, D), bp.reshape(1, 1))


# speedup vs baseline: 7.4419x; 1.7859x over previous
import functools

import jax
import jax.numpy as jnp
from jax import lax
from jax.experimental import pallas as pl
from jax.experimental.pallas import tpu as pltpu
from jax.experimental.pallas import tpu_sc as plsc

N = 10000
E = 320000
D = 128
NC = 2
NS = 16
NW = NC * NS
EPW = E // NW
C = 80
NCHUNK = EPW // C
NP = 10240
RPT = NP // NS
ZR = 128
DEGW = 128

def _fill_vmem(ref, rows, width, value):
    def body(i, _):
        for j in range(width // 16):
            ref[i, pl.ds(16 * j, 16)] = jnp.full((16,), value, jnp.float32)
        return 0
    lax.fori_loop(0, rows, body, 0, unroll=False)


@functools.cache
def _sc_kernels():
    mesh = plsc.VectorSubcoreMesh(core_axis_name="c", subcore_axis_name="s",
                                  num_cores=NC, num_subcores=NS)

    @functools.partial(
        pl.kernel,
        mesh=mesh,
        out_type=jax.ShapeDtypeStruct((NC, NP, DEGW), jnp.float32),
        scratch_types=[
            pltpu.VMEM((C,), jnp.int32),
            pltpu.VMEM((C, DEGW), jnp.float32),
            pltpu.VMEM((ZR, DEGW), jnp.float32),
            pltpu.VMEM_SHARED((NP, DEGW), jnp.float32),
        ],
    )
    def deg_kernel(dst_hbm, out_hbm, idx_v, ones_v, zbuf_v, acc_sh):
        cid = lax.axis_index("c")
        sid = lax.axis_index("s")
        wid = cid * NS + sid

        _fill_vmem(ones_v, C, DEGW, 1.0)
        _fill_vmem(zbuf_v, ZR, DEGW, 0.0)
        for k in range(RPT // ZR):
            pltpu.sync_copy(zbuf_v, acc_sh.at[pl.ds(sid * RPT + k * ZR, ZR)])
        plsc.subcore_barrier()

        base = wid * EPW

        def body(i, _):
            pltpu.sync_copy(dst_hbm.at[pl.ds(base + i * C, C)], idx_v)
            pltpu.sync_copy(ones_v, acc_sh.at[idx_v], add=True)
            return 0
        lax.fori_loop(0, NCHUNK, body, 0, unroll=False)

        plsc.subcore_barrier()
        pltpu.sync_copy(acc_sh.at[pl.ds(sid * RPT, RPT)],
                        out_hbm.at[cid, pl.ds(sid * RPT, RPT)])

    @functools.partial(
        pl.kernel,
        mesh=mesh,
        out_type=jax.ShapeDtypeStruct((NC, NP, D), jnp.float32),
        scratch_types=[
            [pltpu.VMEM((C,), jnp.int32)] * 4,
            [pltpu.VMEM((C,), jnp.int32)] * 4,
            [pltpu.VMEM((C, D), jnp.float32)] * 4,
            pltpu.VMEM_SHARED((NP, D), jnp.float32),
            [pltpu.SemaphoreType.DMA] * 4,
            [pltpu.SemaphoreType.DMA] * 4,
            [pltpu.SemaphoreType.DMA] * 4,
        ],
    )
    def spmm_kernel(g_hbm, src_hbm, dst_hbm, out_hbm,
                    sq, dq, rows, y_sh, isem, gsem, ssem):
        cid = lax.axis_index("c")
        sid = lax.axis_index("s")
        wid = cid * NS + sid
        base = wid * EPW

        def idx_copy(c_off, j):
            off = base + c_off * C
            pltpu.async_copy(src_hbm.at[pl.ds(off, C)], sq[j], isem[j])
            pltpu.async_copy(dst_hbm.at[pl.ds(off, C)], dq[j], isem[j])

        def idx_wait(j):
            pltpu.make_async_copy(
                src_hbm.at[pl.ds(base, C)], sq[j], isem[j]).wait()
            pltpu.make_async_copy(
                dst_hbm.at[pl.ds(base, C)], dq[j], isem[j]).wait()

        def gather(j):
            return pltpu.async_copy(g_hbm.at[sq[j]], rows[j], gsem[j])

        def scat(j):
            return pltpu.make_async_copy(rows[j], y_sh.at[dq[j]], ssem[j])

        idx_copy(0, 0)
        idx_copy(1, 1)
        _fill_vmem(rows[0], C, D, 0.0)
        for k in range(RPT // C):
            pltpu.sync_copy(rows[0], y_sh.at[pl.ds(sid * RPT + k * C, C)])
        plsc.subcore_barrier()
        idx_wait(0)
        gather(0)

        def body(k, _):
            for j in range(4):
                c = 4 * k + j
                pltpu.make_async_copy(
                    g_hbm.at[sq[j]], rows[j], gsem[j]).wait()
                pltpu.async_copy(rows[j], y_sh.at[dq[j]], ssem[j],
                                 add=True)
                if j < 2:
                    @pl.when(k > 0)
                    def _():
                        scat((j + 2) % 4).wait()
                else:
                    scat((j + 2) % 4).wait()
                if j == 3:
                    @pl.when(k < (NCHUNK - 1) // 4 - 1)
                    def _():
                        idx_copy(c + 2, (j + 2) % 4)
                else:
                    idx_copy(c + 2, (j + 2) % 4)
                idx_wait((j + 1) % 4)
                gather((j + 1) % 4)
            return 0
        lax.fori_loop(0, (NCHUNK - 1) // 4, body, 0, unroll=False)

        scat(2).wait()
        scat(3).wait()
        pltpu.make_async_copy(g_hbm.at[sq[0]], rows[0], gsem[0]).wait()
        pltpu.async_copy(rows[0], y_sh.at[dq[0]], ssem[0], add=True)
        scat(0).wait()

        plsc.subcore_barrier()
        pltpu.sync_copy(y_sh.at[pl.ds(sid * RPT, RPT)],
                        out_hbm.at[cid, pl.ds(sid * RPT, RPT)])

    return deg_kernel, spmm_kernel


R = 400
GRID = N // R
_F32MAX = 3.4028234663852886e38


def _prep_body(deg2_ref, x_ref, normb_ref, normb2_ref, g1_ref, h0_ref):
    deg = deg2_ref[0, :, 0:1] + deg2_ref[1, :, 0:1]
    nrm = lax.rsqrt(jnp.maximum(deg, 1.0))
    nb = jnp.broadcast_to(nrm, (R, D))
    xb = x_ref[...]
    h0 = jnp.where(jnp.isnan(xb), 0.0, xb)
    h0 = jnp.clip(h0, -_F32MAX, _F32MAX)
    normb_ref[...] = nb
    normb2_ref[...] = nb * nb
    g1_ref[...] = nb * h0
    h0_ref[...] = h0


_prep_call = pl.pallas_call(
    _prep_body,
    grid=(GRID,),
    in_specs=[
        pl.BlockSpec((NC, R, DEGW), lambda i: (0, i, 0)),
        pl.BlockSpec((R, D), lambda i: (i, 0)),
    ],
    out_specs=[pl.BlockSpec((R, D), lambda i: (i, 0))] * 4,
    out_shape=[jax.ShapeDtypeStruct((N, D), jnp.float32)] * 4,
)


def _scale_body(ypair_ref, nb2_ref, ysum_ref, g2_ref):
    y = ypair_ref[0] + ypair_ref[1]
    ysum_ref[...] = y
    g2_ref[...] = nb2_ref[...] * y


_scale_call = pl.pallas_call(
    _scale_body,
    grid=(GRID,),
    in_specs=[
        pl.BlockSpec((NC, R, D), lambda i: (0, i, 0)),
        pl.BlockSpec((R, D), lambda i: (i, 0)),
    ],
    out_specs=[pl.BlockSpec((R, D), lambda i: (i, 0))] * 2,
    out_shape=[jax.ShapeDtypeStruct((N, D), jnp.float32)] * 2,
)


def _layer_body(h_ref, y1_ref, y2p_ref, nb_ref, w_ref, b_ref, hn_ref, gn_ref):
    y2 = y2p_ref[0] + y2p_ref[1]
    dot = functools.partial(jnp.dot, preferred_element_type=jnp.float32,
                            precision=lax.Precision.HIGHEST)
    part = dot(y1_ref[...], w_ref[1]) + dot(y2, w_ref[2])
    out = dot(h_ref[...], w_ref[0]) + nb_ref[...] * part + b_ref[...]
    hn = jnp.maximum(out, jnp.float32(0.0))
    hn_ref[...] = hn
    gn_ref[...] = nb_ref[...] * hn


_layer_call = pl.pallas_call(
    _layer_body,
    grid=(GRID,),
    in_specs=[
        pl.BlockSpec((R, D), lambda i: (i, 0)),
        pl.BlockSpec((R, D), lambda i: (i, 0)),
        pl.BlockSpec((NC, R, D), lambda i: (0, i, 0)),
        pl.BlockSpec((R, D), lambda i: (i, 0)),
        pl.BlockSpec((3, D, D), lambda i: (0, 0, 0)),
        pl.BlockSpec((1, D), lambda i: (0, 0)),
    ],
    out_specs=[pl.BlockSpec((R, D), lambda i: (i, 0))] * 2,
    out_shape=[jax.ShapeDtypeStruct((N, D), jnp.float32)] * 2,
)


def _final_body(h_ref, wpt_ref, bp_ref, out_ref, acc_ref):
    i = pl.program_id(0)

    @pl.when(i == 0)
    def _():
        acc_ref[...] = jnp.zeros_like(acc_ref)
        out_ref[...] = jnp.zeros((1, 1), jnp.float32)

    acc_ref[...] += h_ref[...].reshape(R // 8, 8, D).sum(axis=0)

    @pl.when(i == GRID - 1)
    def _():
        tot = acc_ref[...].sum(axis=0, keepdims=True)
        val = jnp.sum(tot * wpt_ref[...]) / N + bp_ref[0, 0]
        out_ref[...] = val.reshape(1, 1)


_final_call = pl.pallas_call(
    _final_body,
    grid=(GRID,),
    in_specs=[
        pl.BlockSpec((R, D), lambda i: (i, 0)),
        pl.BlockSpec((1, D), lambda i: (0, 0)),
        pl.BlockSpec((1, 1), lambda i: (0, 0)),
    ],
    out_specs=pl.BlockSpec((1, 1), lambda i: (0, 0)),
    out_shape=jax.ShapeDtypeStruct((1, 1), jnp.float32),
    scratch_shapes=[pltpu.VMEM((8, D), jnp.float32)],
)


def kernel(x, edge_index, W1, b1, W2, b2, W3, b3, W4, b4, W5, b5, Wp, bp):
    src = edge_index[0]
    dst = edge_index[1]
    deg_kernel, spmm_kernel = _sc_kernels()

    deg2 = deg_kernel(dst)
    normb, normb2, g, h = _prep_call(deg2, x)

    for W, b in ((W1, b1), (W2, b2), (W3, b3), (W4, b4), (W5, b5)):
        y1p = spmm_kernel(g, src, dst)
        y1, g2 = _scale_call(y1p, normb2)
        y2p = spmm_kernel(g2, src, dst)
        h, g = _layer_call(h, y1, y2p, normb, W.reshape(3, D, D),
                           b.reshape(1, D))

    return _final_call(h, Wp.reshape(1, D), bp.reshape(1, 1))


# speedup vs baseline: 9.4633x; 1.2716x over previous
import functools

import jax
import jax.numpy as jnp
from jax import lax
from jax.experimental import pallas as pl
from jax.experimental.pallas import tpu as pltpu
from jax.experimental.pallas import tpu_sc as plsc

N = 10000
E = 320000
D = 128
NC = 2
NS = 16
NW = NC * NS
EPW = E // NW
C = 80
NCHUNK = EPW // C
NP = 10240
RPT = NP // NS
ZR = 128
DEGW = 128

def _fill_vmem(ref, rows, width, value):
    def body(i, _):
        for j in range(width // 16):
            ref[i, pl.ds(16 * j, 16)] = jnp.full((16,), value, jnp.float32)
        return 0
    lax.fori_loop(0, rows, body, 0, unroll=False)


@functools.cache
def _sc_kernels():
    mesh = plsc.VectorSubcoreMesh(core_axis_name="c", subcore_axis_name="s",
                                  num_cores=NC, num_subcores=NS)

    @functools.partial(
        pl.kernel,
        mesh=mesh,
        out_type=jax.ShapeDtypeStruct((NC, NP, DEGW), jnp.float32),
        scratch_types=[
            pltpu.VMEM((C,), jnp.int32),
            pltpu.VMEM((C, DEGW), jnp.float32),
            pltpu.VMEM((ZR, DEGW), jnp.float32),
            pltpu.VMEM_SHARED((NP, DEGW), jnp.float32),
        ],
    )
    def deg_kernel(dst_hbm, out_hbm, idx_v, ones_v, zbuf_v, acc_sh):
        cid = lax.axis_index("c")
        sid = lax.axis_index("s")
        wid = cid * NS + sid

        _fill_vmem(ones_v, C, DEGW, 1.0)
        _fill_vmem(zbuf_v, ZR, DEGW, 0.0)
        for k in range(RPT // ZR):
            pltpu.sync_copy(zbuf_v, acc_sh.at[pl.ds(sid * RPT + k * ZR, ZR)])
        plsc.subcore_barrier()

        base = wid * EPW

        def body(i, _):
            pltpu.sync_copy(dst_hbm.at[pl.ds(base + i * C, C)], idx_v)
            pltpu.sync_copy(ones_v, acc_sh.at[idx_v], add=True)
            return 0
        lax.fori_loop(0, NCHUNK, body, 0, unroll=False)

        plsc.subcore_barrier()
        pltpu.sync_copy(acc_sh.at[pl.ds(sid * RPT, RPT)],
                        out_hbm.at[cid, pl.ds(sid * RPT, RPT)])

    @functools.partial(
        pl.kernel,
        mesh=mesh,
        out_type=jax.ShapeDtypeStruct((NC, NP, D), jnp.float32),
        scratch_types=[
            [pltpu.VMEM((2, C), jnp.int32)] * 8,
            [pltpu.VMEM((C, D), jnp.float32)] * 4,
            pltpu.VMEM_SHARED((NP, D), jnp.float32),
            [pltpu.SemaphoreType.DMA] * 8,
            [pltpu.SemaphoreType.DMA] * 4,
            [pltpu.SemaphoreType.DMA] * 4,
        ],
    )
    def spmm_kernel(g_hbm, sd_hbm, out_hbm,
                    q, rows, y_sh, isem, gsem, ssem):
        cid = lax.axis_index("c")
        sid = lax.axis_index("s")
        wid = cid * NS + sid
        cbase = wid * NCHUNK

        def idx_copy(c, jq):
            pltpu.async_copy(sd_hbm.at[cbase + c], q[jq], isem[jq])

        def idx_wait(jq):
            pltpu.make_async_copy(sd_hbm.at[cbase], q[jq], isem[jq]).wait()

        def gather(jq, jr):
            pltpu.async_copy(g_hbm.at[q[jq].at[0]], rows[jr], gsem[jr])

        def gather_wait(jr):
            pltpu.make_async_copy(g_hbm.at[q[0].at[0]], rows[jr],
                                  gsem[jr]).wait()

        def scatter(jq, jr):
            pltpu.async_copy(rows[jr], y_sh.at[q[jq].at[1]], ssem[jr],
                             add=True)

        def scatter_wait(jr):
            pltpu.make_async_copy(rows[jr], y_sh.at[q[0].at[1]],
                                  ssem[jr]).wait()

        for c in range(6):
            idx_copy(c, c)
        _fill_vmem(rows[0], C, D, 0.0)
        for k in range(RPT // C):
            pltpu.sync_copy(rows[0], y_sh.at[pl.ds(sid * RPT + k * C, C)])
        plsc.subcore_barrier()
        idx_wait(0)
        gather(0, 0)
        idx_wait(1)
        gather(1, 1)

        NK = NCHUNK // 8
        def body(k, _):
            for j in range(8):
                c = 8 * k + j
                gather_wait(j % 4)
                scatter(j, j % 4)
                if j < 2:
                    @pl.when(k > 0)
                    def _():
                        scatter_wait((j + 2) % 4)
                else:
                    scatter_wait((j + 2) % 4)
                if j == 7:
                    @pl.when(k < NK - 1)
                    def _():
                        idx_copy(c + 6, (j + 6) % 8)
                else:
                    idx_copy(c + 6, (j + 6) % 8)
                idx_wait((j + 2) % 8)
                gather((j + 2) % 8, (j + 2) % 4)
            return 0
        lax.fori_loop(0, NK, body, 0, unroll=False)

        b0 = 8 * NK
        gather_wait(0); scatter(0, 0); scatter_wait(2)
        idx_wait(2); gather(2, 2)
        gather_wait(1); scatter(1, 1); scatter_wait(3)
        idx_wait(3); gather(3, 3)
        gather_wait(2); scatter(2, 2); scatter_wait(0)
        idx_wait(4); gather(4, 0)
        gather_wait(3); scatter(3, 3); scatter_wait(1)
        gather_wait(0); scatter(4, 0); scatter_wait(2)
        scatter_wait(3)
        scatter_wait(0)
        del b0

        plsc.subcore_barrier()
        pltpu.sync_copy(y_sh.at[pl.ds(sid * RPT, RPT)],
                        out_hbm.at[cid, pl.ds(sid * RPT, RPT)])

    return deg_kernel, spmm_kernel


R = 400
GRID = N // R
_F32MAX = 3.4028234663852886e38


def _prep_body(deg2_ref, x_ref, normb_ref, normb2_ref, g1_ref, h0_ref):
    deg = deg2_ref[0, :, 0:1] + deg2_ref[1, :, 0:1]
    nrm = lax.rsqrt(jnp.maximum(deg, 1.0))
    nb = jnp.broadcast_to(nrm, (R, D))
    xb = x_ref[...]
    h0 = jnp.where(jnp.isnan(xb), 0.0, xb)
    h0 = jnp.clip(h0, -_F32MAX, _F32MAX)
    normb_ref[...] = nb
    normb2_ref[...] = nb * nb
    g1_ref[...] = nb * h0
    h0_ref[...] = h0


_prep_call = pl.pallas_call(
    _prep_body,
    grid=(GRID,),
    in_specs=[
        pl.BlockSpec((NC, R, DEGW), lambda i: (0, i, 0)),
        pl.BlockSpec((R, D), lambda i: (i, 0)),
    ],
    out_specs=[pl.BlockSpec((R, D), lambda i: (i, 0))] * 4,
    out_shape=[jax.ShapeDtypeStruct((N, D), jnp.float32)] * 4,
)


def _scale_body(ypair_ref, nb2_ref, ysum_ref, g2_ref):
    y = ypair_ref[0] + ypair_ref[1]
    ysum_ref[...] = y
    g2_ref[...] = nb2_ref[...] * y


_scale_call = pl.pallas_call(
    _scale_body,
    grid=(GRID,),
    in_specs=[
        pl.BlockSpec((NC, R, D), lambda i: (0, i, 0)),
        pl.BlockSpec((R, D), lambda i: (i, 0)),
    ],
    out_specs=[pl.BlockSpec((R, D), lambda i: (i, 0))] * 2,
    out_shape=[jax.ShapeDtypeStruct((N, D), jnp.float32)] * 2,
)


def _layer_body(h_ref, y1_ref, y2p_ref, nb_ref, w_ref, b_ref, hn_ref, gn_ref):
    y2 = y2p_ref[0] + y2p_ref[1]
    dot = functools.partial(jnp.dot, preferred_element_type=jnp.float32,
                            precision=lax.Precision.HIGHEST)
    part = dot(y1_ref[...], w_ref[1]) + dot(y2, w_ref[2])
    out = dot(h_ref[...], w_ref[0]) + nb_ref[...] * part + b_ref[...]
    hn = jnp.maximum(out, jnp.float32(0.0))
    hn_ref[...] = hn
    gn_ref[...] = nb_ref[...] * hn


_layer_call = pl.pallas_call(
    _layer_body,
    grid=(GRID,),
    in_specs=[
        pl.BlockSpec((R, D), lambda i: (i, 0)),
        pl.BlockSpec((R, D), lambda i: (i, 0)),
        pl.BlockSpec((NC, R, D), lambda i: (0, i, 0)),
        pl.BlockSpec((R, D), lambda i: (i, 0)),
        pl.BlockSpec((3, D, D), lambda i: (0, 0, 0)),
        pl.BlockSpec((1, D), lambda i: (0, 0)),
    ],
    out_specs=[pl.BlockSpec((R, D), lambda i: (i, 0))] * 2,
    out_shape=[jax.ShapeDtypeStruct((N, D), jnp.float32)] * 2,
)


def _final_body(h_ref, wpt_ref, bp_ref, out_ref, acc_ref):
    i = pl.program_id(0)

    @pl.when(i == 0)
    def _():
        acc_ref[...] = jnp.zeros_like(acc_ref)
        out_ref[...] = jnp.zeros((1, 1), jnp.float32)

    acc_ref[...] += h_ref[...].reshape(R // 8, 8, D).sum(axis=0)

    @pl.when(i == GRID - 1)
    def _():
        tot = acc_ref[...].sum(axis=0, keepdims=True)
        val = jnp.sum(tot * wpt_ref[...]) / N + bp_ref[0, 0]
        out_ref[...] = val.reshape(1, 1)


_final_call = pl.pallas_call(
    _final_body,
    grid=(GRID,),
    in_specs=[
        pl.BlockSpec((R, D), lambda i: (i, 0)),
        pl.BlockSpec((1, D), lambda i: (0, 0)),
        pl.BlockSpec((1, 1), lambda i: (0, 0)),
    ],
    out_specs=pl.BlockSpec((1, 1), lambda i: (0, 0)),
    out_shape=jax.ShapeDtypeStruct((1, 1), jnp.float32),
    scratch_shapes=[pltpu.VMEM((8, D), jnp.float32)],
)


def kernel(x, edge_index, W1, b1, W2, b2, W3, b3, W4, b4, W5, b5, Wp, bp):
    src = edge_index[0]
    dst = edge_index[1]
    sd = jnp.stack([src.reshape(E // C, C), dst.reshape(E // C, C)], axis=1)
    deg_kernel, spmm_kernel = _sc_kernels()

    deg2 = deg_kernel(dst)
    normb, normb2, g, h = _prep_call(deg2, x)

    for W, b in ((W1, b1), (W2, b2), (W3, b3), (W4, b4), (W5, b5)):
        y1p = spmm_kernel(g, sd)
        y1, g2 = _scale_call(y1p, normb2)
        y2p = spmm_kernel(g2, sd)
        h, g = _layer_call(h, y1, y2p, normb, W.reshape(3, D, D),
                           b.reshape(1, D))

    return _final_call(h, Wp.reshape(1, D), bp.reshape(1, 1))


# speedup vs baseline: 9.7778x; 1.0332x over previous
import functools

import jax
import jax.numpy as jnp
from jax import lax
from jax.experimental import pallas as pl
from jax.experimental.pallas import tpu as pltpu
from jax.experimental.pallas import tpu_sc as plsc

N = 10000
E = 320000
D = 128
NC = 2
NS = 16
NW = NC * NS
EPW = E // NW
C = 80
NCHUNK = EPW // C
NP = 10240
RPT = NP // NS
ZR = 128
DEGW = 128

def _fill_vmem(ref, rows, width, value):
    def body(i, _):
        for j in range(width // 16):
            ref[i, pl.ds(16 * j, 16)] = jnp.full((16,), value, jnp.float32)
        return 0
    lax.fori_loop(0, rows, body, 0, unroll=False)


@functools.cache
def _sc_kernels():
    mesh = plsc.VectorSubcoreMesh(core_axis_name="c", subcore_axis_name="s",
                                  num_cores=NC, num_subcores=NS)

    @functools.partial(
        pl.kernel,
        mesh=mesh,
        out_type=jax.ShapeDtypeStruct((NC, NP, DEGW), jnp.float32),
        scratch_types=[
            [pltpu.VMEM((2, C), jnp.int32)] * 8,
            pltpu.VMEM((C, DEGW), jnp.float32),
            pltpu.VMEM((C, DEGW), jnp.float32),
            pltpu.VMEM_SHARED((NP, DEGW), jnp.float32),
            [pltpu.SemaphoreType.DMA] * 8,
            [pltpu.SemaphoreType.DMA] * 4,
        ],
    )
    def deg_kernel(sd_hbm, out_hbm, q, ones_v, zbuf_v, acc_sh, isem, ssem):
        cid = lax.axis_index("c")
        sid = lax.axis_index("s")
        wid = cid * NS + sid
        cbase = wid * NCHUNK

        def idx_copy(c, jq):
            pltpu.async_copy(sd_hbm.at[cbase + c], q[jq], isem[jq])

        def idx_wait(jq):
            pltpu.make_async_copy(sd_hbm.at[cbase], q[jq], isem[jq]).wait()

        def scatter(jq, jr):
            pltpu.async_copy(ones_v, acc_sh.at[q[jq].at[1]], ssem[jr],
                             add=True)

        def scatter_wait(jr):
            pltpu.make_async_copy(ones_v, acc_sh.at[q[0].at[1]],
                                  ssem[jr]).wait()

        for c in range(6):
            idx_copy(c, c)
        _fill_vmem(ones_v, C, DEGW, 1.0)
        _fill_vmem(zbuf_v, C, DEGW, 0.0)
        for k in range(RPT // C):
            pltpu.sync_copy(zbuf_v, acc_sh.at[pl.ds(sid * RPT + k * C, C)])
        plsc.subcore_barrier()

        NK = NCHUNK // 8
        def body(k, _):
            for j in range(8):
                c = 8 * k + j
                idx_wait(j % 8)
                scatter(j, j % 4)
                if j < 2:
                    @pl.when(k > 0)
                    def _():
                        scatter_wait((j + 2) % 4)
                else:
                    scatter_wait((j + 2) % 4)
                if j == 7:
                    @pl.when(k < NK - 1)
                    def _():
                        idx_copy(c + 6, (j + 6) % 8)
                else:
                    idx_copy(c + 6, (j + 6) % 8)
            return 0
        lax.fori_loop(0, NK, body, 0, unroll=False)

        idx_wait(0); scatter(0, 0); scatter_wait(2)
        idx_wait(1); scatter(1, 1); scatter_wait(3)
        idx_wait(2); scatter(2, 2); scatter_wait(0)
        idx_wait(3); scatter(3, 3); scatter_wait(1)
        idx_wait(4); scatter(4, 0); scatter_wait(2)
        scatter_wait(3)
        scatter_wait(0)

        plsc.subcore_barrier()
        pltpu.sync_copy(acc_sh.at[pl.ds(sid * RPT, RPT)],
                        out_hbm.at[cid, pl.ds(sid * RPT, RPT)])

    @functools.partial(
        pl.kernel,
        mesh=mesh,
        out_type=jax.ShapeDtypeStruct((NC, NP, D), jnp.float32),
        scratch_types=[
            [pltpu.VMEM((2, C), jnp.int32)] * 8,
            [pltpu.VMEM((C, D), jnp.float32)] * 4,
            pltpu.VMEM_SHARED((NP, D), jnp.float32),
            [pltpu.SemaphoreType.DMA] * 8,
            [pltpu.SemaphoreType.DMA] * 4,
            [pltpu.SemaphoreType.DMA] * 4,
        ],
    )
    def spmm_kernel(g_hbm, sd_hbm, out_hbm,
                    q, rows, y_sh, isem, gsem, ssem):
        cid = lax.axis_index("c")
        sid = lax.axis_index("s")
        wid = cid * NS + sid
        cbase = wid * NCHUNK

        def idx_copy(c, jq):
            pltpu.async_copy(sd_hbm.at[cbase + c], q[jq], isem[jq])

        def idx_wait(jq):
            pltpu.make_async_copy(sd_hbm.at[cbase], q[jq], isem[jq]).wait()

        def gather(jq, jr):
            pltpu.async_copy(g_hbm.at[q[jq].at[0]], rows[jr], gsem[jr])

        def gather_wait(jr):
            pltpu.make_async_copy(g_hbm.at[q[0].at[0]], rows[jr],
                                  gsem[jr]).wait()

        def scatter(jq, jr):
            pltpu.async_copy(rows[jr], y_sh.at[q[jq].at[1]], ssem[jr],
                             add=True)

        def scatter_wait(jr):
            pltpu.make_async_copy(rows[jr], y_sh.at[q[0].at[1]],
                                  ssem[jr]).wait()

        for c in range(6):
            idx_copy(c, c)
        _fill_vmem(rows[0], C, D, 0.0)
        for k in range(RPT // C):
            pltpu.sync_copy(rows[0], y_sh.at[pl.ds(sid * RPT + k * C, C)])
        plsc.subcore_barrier()
        idx_wait(0)
        gather(0, 0)
        idx_wait(1)
        gather(1, 1)

        NK = NCHUNK // 8
        def body(k, _):
            for j in range(8):
                c = 8 * k + j
                gather_wait(j % 4)
                scatter(j, j % 4)
                if j < 2:
                    @pl.when(k > 0)
                    def _():
                        scatter_wait((j + 2) % 4)
                else:
                    scatter_wait((j + 2) % 4)
                if j == 7:
                    @pl.when(k < NK - 1)
                    def _():
                        idx_copy(c + 6, (j + 6) % 8)
                else:
                    idx_copy(c + 6, (j + 6) % 8)
                idx_wait((j + 2) % 8)
                gather((j + 2) % 8, (j + 2) % 4)
            return 0
        lax.fori_loop(0, NK, body, 0, unroll=False)

        b0 = 8 * NK
        gather_wait(0); scatter(0, 0); scatter_wait(2)
        idx_wait(2); gather(2, 2)
        gather_wait(1); scatter(1, 1); scatter_wait(3)
        idx_wait(3); gather(3, 3)
        gather_wait(2); scatter(2, 2); scatter_wait(0)
        idx_wait(4); gather(4, 0)
        gather_wait(3); scatter(3, 3); scatter_wait(1)
        gather_wait(0); scatter(4, 0); scatter_wait(2)
        scatter_wait(3)
        scatter_wait(0)
        del b0

        plsc.subcore_barrier()
        pltpu.sync_copy(y_sh.at[pl.ds(sid * RPT, RPT)],
                        out_hbm.at[cid, pl.ds(sid * RPT, RPT)])

    return deg_kernel, spmm_kernel


R = 400
GRID = N // R
_F32MAX = 3.4028234663852886e38


def _prep_body(deg2_ref, x_ref, normb_ref, normb2_ref, g1_ref, h0_ref):
    deg = deg2_ref[0, :, 0:1] + deg2_ref[1, :, 0:1]
    nrm = lax.rsqrt(jnp.maximum(deg, 1.0))
    nb = jnp.broadcast_to(nrm, (R, D))
    xb = x_ref[...]
    h0 = jnp.where(jnp.isnan(xb), 0.0, xb)
    h0 = jnp.clip(h0, -_F32MAX, _F32MAX)
    normb_ref[...] = nb
    normb2_ref[...] = nb * nb
    g1_ref[...] = nb * h0
    h0_ref[...] = h0


_prep_call = pl.pallas_call(
    _prep_body,
    grid=(GRID,),
    in_specs=[
        pl.BlockSpec((NC, R, DEGW), lambda i: (0, i, 0)),
        pl.BlockSpec((R, D), lambda i: (i, 0)),
    ],
    out_specs=[pl.BlockSpec((R, D), lambda i: (i, 0))] * 4,
    out_shape=[jax.ShapeDtypeStruct((N, D), jnp.float32)] * 4,
)


def _scale_body(ypair_ref, nb2_ref, ysum_ref, g2_ref):
    y = ypair_ref[0] + ypair_ref[1]
    ysum_ref[...] = y
    g2_ref[...] = nb2_ref[...] * y


_scale_call = pl.pallas_call(
    _scale_body,
    grid=(GRID,),
    in_specs=[
        pl.BlockSpec((NC, R, D), lambda i: (0, i, 0)),
        pl.BlockSpec((R, D), lambda i: (i, 0)),
    ],
    out_specs=[pl.BlockSpec((R, D), lambda i: (i, 0))] * 2,
    out_shape=[jax.ShapeDtypeStruct((N, D), jnp.float32)] * 2,
)


def _layer_body(h_ref, y1_ref, y2p_ref, nb_ref, w_ref, b_ref, hn_ref, gn_ref):
    y2 = y2p_ref[0] + y2p_ref[1]
    dot = functools.partial(jnp.dot, preferred_element_type=jnp.float32,
                            precision=lax.Precision.HIGHEST)
    part = dot(y1_ref[...], w_ref[1]) + dot(y2, w_ref[2])
    out = dot(h_ref[...], w_ref[0]) + nb_ref[...] * part + b_ref[...]
    hn = jnp.maximum(out, jnp.float32(0.0))
    hn_ref[...] = hn
    gn_ref[...] = nb_ref[...] * hn


_layer_call = pl.pallas_call(
    _layer_body,
    grid=(GRID,),
    in_specs=[
        pl.BlockSpec((R, D), lambda i: (i, 0)),
        pl.BlockSpec((R, D), lambda i: (i, 0)),
        pl.BlockSpec((NC, R, D), lambda i: (0, i, 0)),
        pl.BlockSpec((R, D), lambda i: (i, 0)),
        pl.BlockSpec((3, D, D), lambda i: (0, 0, 0)),
        pl.BlockSpec((1, D), lambda i: (0, 0)),
    ],
    out_specs=[pl.BlockSpec((R, D), lambda i: (i, 0))] * 2,
    out_shape=[jax.ShapeDtypeStruct((N, D), jnp.float32)] * 2,
)


def _final_body(h_ref, wpt_ref, bp_ref, out_ref, acc_ref):
    i = pl.program_id(0)

    @pl.when(i == 0)
    def _():
        acc_ref[...] = jnp.zeros_like(acc_ref)
        out_ref[...] = jnp.zeros((1, 1), jnp.float32)

    acc_ref[...] += h_ref[...].reshape(R // 8, 8, D).sum(axis=0)

    @pl.when(i == GRID - 1)
    def _():
        tot = acc_ref[...].sum(axis=0, keepdims=True)
        val = jnp.sum(tot * wpt_ref[...]) / N + bp_ref[0, 0]
        out_ref[...] = val.reshape(1, 1)


_final_call = pl.pallas_call(
    _final_body,
    grid=(GRID,),
    in_specs=[
        pl.BlockSpec((R, D), lambda i: (i, 0)),
        pl.BlockSpec((1, D), lambda i: (0, 0)),
        pl.BlockSpec((1, 1), lambda i: (0, 0)),
    ],
    out_specs=pl.BlockSpec((1, 1), lambda i: (0, 0)),
    out_shape=jax.ShapeDtypeStruct((1, 1), jnp.float32),
    scratch_shapes=[pltpu.VMEM((8, D), jnp.float32)],
)


def kernel(x, edge_index, W1, b1, W2, b2, W3, b3, W4, b4, W5, b5, Wp, bp):
    src = edge_index[0]
    dst = edge_index[1]
    sd = jnp.stack([src.reshape(E // C, C), dst.reshape(E // C, C)], axis=1)
    deg_kernel, spmm_kernel = _sc_kernels()

    deg2 = deg_kernel(sd)
    normb, normb2, g, h = _prep_call(deg2, x)

    for W, b in ((W1, b1), (W2, b2), (W3, b3), (W4, b4), (W5, b5)):
        y1p = spmm_kernel(g, sd)
        y1, g2 = _scale_call(y1p, normb2)
        y2p = spmm_kernel(g2, sd)
        h, g = _layer_call(h, y1, y2p, normb, W.reshape(3, D, D),
                           b.reshape(1, D))

    return _final_call(h, Wp.reshape(1, D), bp.reshape(1, 1))


# speedup vs baseline: 9.8863x; 1.0111x over previous
import functools

import jax
import jax.numpy as jnp
from jax import lax
from jax.experimental import pallas as pl
from jax.experimental.pallas import tpu as pltpu
from jax.experimental.pallas import tpu_sc as plsc

N = 10000
E = 320000
D = 128
NC = 2
NS = 16
NW = NC * NS
EPW = E // NW
C = 80
NCHUNK = EPW // C
NP = 10240
RPT = NP // NS
ZR = 128
DEGW = 128

def _fill_vmem(ref, rows, width, value):
    def body(i, _):
        for j in range(width // 16):
            ref[i, pl.ds(16 * j, 16)] = jnp.full((16,), value, jnp.float32)
        return 0
    lax.fori_loop(0, rows, body, 0, unroll=False)


@functools.cache
def _sc_kernels():
    mesh = plsc.VectorSubcoreMesh(core_axis_name="c", subcore_axis_name="s",
                                  num_cores=NC, num_subcores=NS)

    @functools.partial(
        pl.kernel,
        mesh=mesh,
        out_type=jax.ShapeDtypeStruct((NC, NP, DEGW), jnp.float32),
        scratch_types=[
            [pltpu.VMEM((2, C), jnp.int32)] * 8,
            pltpu.VMEM((C, DEGW), jnp.float32),
            pltpu.VMEM((C, DEGW), jnp.float32),
            pltpu.VMEM_SHARED((NP, DEGW), jnp.float32),
            [pltpu.SemaphoreType.DMA] * 8,
            [pltpu.SemaphoreType.DMA] * 4,
        ],
    )
    def deg_kernel(sd_hbm, out_hbm, q, ones_v, zbuf_v, acc_sh, isem, ssem):
        cid = lax.axis_index("c")
        sid = lax.axis_index("s")
        wid = cid * NS + sid
        cbase = wid * NCHUNK

        def idx_copy(c, jq):
            pltpu.async_copy(sd_hbm.at[cbase + c], q[jq], isem[jq])

        def idx_wait(jq):
            pltpu.make_async_copy(sd_hbm.at[cbase], q[jq], isem[jq]).wait()

        def scatter(jq, jr):
            pltpu.async_copy(ones_v, acc_sh.at[q[jq].at[1]], ssem[jr],
                             add=True)

        def scatter_wait(jr):
            pltpu.make_async_copy(ones_v, acc_sh.at[q[0].at[1]],
                                  ssem[jr]).wait()

        for c in range(6):
            idx_copy(c, c)
        _fill_vmem(ones_v, C, DEGW, 1.0)
        _fill_vmem(zbuf_v, C, DEGW, 0.0)
        for k in range(RPT // C):
            pltpu.sync_copy(zbuf_v, acc_sh.at[pl.ds(sid * RPT + k * C, C)])
        plsc.subcore_barrier()

        NK = NCHUNK // 8
        def body(k, _):
            for j in range(8):
                c = 8 * k + j
                idx_wait(j % 8)
                scatter(j, j % 4)
                if j < 2:
                    @pl.when(k > 0)
                    def _():
                        scatter_wait((j + 2) % 4)
                else:
                    scatter_wait((j + 2) % 4)
                if j == 7:
                    @pl.when(k < NK - 1)
                    def _():
                        idx_copy(c + 6, (j + 6) % 8)
                else:
                    idx_copy(c + 6, (j + 6) % 8)
            return 0
        lax.fori_loop(0, NK, body, 0, unroll=False)

        idx_wait(0); scatter(0, 0); scatter_wait(2)
        idx_wait(1); scatter(1, 1); scatter_wait(3)
        idx_wait(2); scatter(2, 2); scatter_wait(0)
        idx_wait(3); scatter(3, 3); scatter_wait(1)
        idx_wait(4); scatter(4, 0); scatter_wait(2)
        scatter_wait(3)
        scatter_wait(0)

        plsc.subcore_barrier()
        pltpu.sync_copy(acc_sh.at[pl.ds(sid * RPT, RPT)],
                        out_hbm.at[cid, pl.ds(sid * RPT, RPT)])

    @functools.partial(
        pl.kernel,
        mesh=mesh,
        out_type=jax.ShapeDtypeStruct((NC, NP, D), jnp.float32),
        scratch_types=[
            [pltpu.VMEM((2, C), jnp.int32)] * 8,
            [pltpu.VMEM((C, D), jnp.float32)] * 4,
            pltpu.VMEM_SHARED((NP, D), jnp.float32),
            [pltpu.SemaphoreType.DMA] * 8,
            [pltpu.SemaphoreType.DMA] * 4,
            [pltpu.SemaphoreType.DMA] * 4,
        ],
    )
    def spmm_kernel(g_hbm, sd_hbm, out_hbm,
                    q, rows, y_sh, isem, gsem, ssem):
        cid = lax.axis_index("c")
        sid = lax.axis_index("s")
        wid = cid * NS + sid
        cbase = wid * NCHUNK

        def idx_copy(c, jq):
            pltpu.async_copy(sd_hbm.at[cbase + c], q[jq], isem[jq])

        def idx_wait(jq):
            pltpu.make_async_copy(sd_hbm.at[cbase], q[jq], isem[jq]).wait()

        def gather(jq, jr):
            pltpu.async_copy(g_hbm.at[q[jq].at[0]], rows[jr], gsem[jr])

        def gather_wait(jr):
            pltpu.make_async_copy(g_hbm.at[q[0].at[0]], rows[jr],
                                  gsem[jr]).wait()

        def scatter(jq, jr):
            pltpu.async_copy(rows[jr], y_sh.at[q[jq].at[1]], ssem[jr],
                             add=True)

        def scatter_wait(jr):
            pltpu.make_async_copy(rows[jr], y_sh.at[q[0].at[1]],
                                  ssem[jr]).wait()

        for c in range(6):
            idx_copy(c, c)
        _fill_vmem(rows[0], C, D, 0.0)
        for k in range(RPT // C):
            pltpu.sync_copy(rows[0], y_sh.at[pl.ds(sid * RPT + k * C, C)])
        plsc.subcore_barrier()
        idx_wait(0)
        gather(0, 0)
        idx_wait(1)
        gather(1, 1)

        NK = NCHUNK // 8
        def body(k, _):
            for j in range(8):
                c = 8 * k + j
                gather_wait(j % 4)
                scatter(j, j % 4)
                if j < 2:
                    @pl.when(k > 0)
                    def _():
                        scatter_wait((j + 2) % 4)
                else:
                    scatter_wait((j + 2) % 4)
                if j == 7:
                    @pl.when(k < NK - 1)
                    def _():
                        idx_copy(c + 6, (j + 6) % 8)
                else:
                    idx_copy(c + 6, (j + 6) % 8)
                idx_wait((j + 2) % 8)
                gather((j + 2) % 8, (j + 2) % 4)
            return 0
        lax.fori_loop(0, NK, body, 0, unroll=False)

        b0 = 8 * NK
        gather_wait(0); scatter(0, 0); scatter_wait(2)
        idx_wait(2); gather(2, 2)
        gather_wait(1); scatter(1, 1); scatter_wait(3)
        idx_wait(3); gather(3, 3)
        gather_wait(2); scatter(2, 2); scatter_wait(0)
        idx_wait(4); gather(4, 0)
        gather_wait(3); scatter(3, 3); scatter_wait(1)
        gather_wait(0); scatter(4, 0); scatter_wait(2)
        scatter_wait(3)
        scatter_wait(0)
        del b0

        plsc.subcore_barrier()
        pltpu.sync_copy(y_sh.at[pl.ds(sid * RPT, RPT)],
                        out_hbm.at[cid, pl.ds(sid * RPT, RPT)])

    return deg_kernel, spmm_kernel


R = 400
GRID = N // R
_F32MAX = 3.4028234663852886e38


def _prep_body(deg2_ref, x_ref, normb_ref, normb2_ref, g1_ref, h0_ref):
    deg = deg2_ref[0, :, 0:1] + deg2_ref[1, :, 0:1]
    nrm = lax.rsqrt(jnp.maximum(deg, 1.0))
    nb = jnp.broadcast_to(nrm, (R, D))
    xb = x_ref[...]
    h0 = jnp.where(jnp.isnan(xb), 0.0, xb)
    h0 = jnp.clip(h0, -_F32MAX, _F32MAX)
    normb_ref[...] = nb
    normb2_ref[...] = nb * nb
    g1_ref[...] = nb * h0
    h0_ref[...] = h0


_prep_call = pl.pallas_call(
    _prep_body,
    grid=(GRID,),
    in_specs=[
        pl.BlockSpec((NC, R, DEGW), lambda i: (0, i, 0)),
        pl.BlockSpec((R, D), lambda i: (i, 0)),
    ],
    out_specs=[pl.BlockSpec((R, D), lambda i: (i, 0))] * 4,
    out_shape=[jax.ShapeDtypeStruct((N, D), jnp.float32)] * 4,
)


def _scale_body(ypair_ref, nb2_ref, ysum_ref, g2_ref):
    y = ypair_ref[0] + ypair_ref[1]
    ysum_ref[...] = y
    g2_ref[...] = nb2_ref[...] * y


_scale_call = pl.pallas_call(
    _scale_body,
    grid=(GRID,),
    in_specs=[
        pl.BlockSpec((NC, R, D), lambda i: (0, i, 0)),
        pl.BlockSpec((R, D), lambda i: (i, 0)),
    ],
    out_specs=[pl.BlockSpec((R, D), lambda i: (i, 0))] * 2,
    out_shape=[jax.ShapeDtypeStruct((N, D), jnp.float32)] * 2,
)


def _layer_body(h_ref, y1_ref, y2p_ref, nb_ref, w_ref, b_ref, hn_ref, gn_ref):
    y2 = y2p_ref[0] + y2p_ref[1]
    dot = functools.partial(jnp.dot, preferred_element_type=jnp.float32,
                            precision=lax.Precision.HIGHEST)
    part = dot(y1_ref[...], w_ref[1]) + dot(y2, w_ref[2])
    out = dot(h_ref[...], w_ref[0]) + nb_ref[...] * part + b_ref[...]
    hn = jnp.maximum(out, jnp.float32(0.0))
    hn_ref[...] = hn
    gn_ref[...] = nb_ref[...] * hn


_layer_call = pl.pallas_call(
    _layer_body,
    grid=(GRID,),
    in_specs=[
        pl.BlockSpec((R, D), lambda i: (i, 0)),
        pl.BlockSpec((R, D), lambda i: (i, 0)),
        pl.BlockSpec((NC, R, D), lambda i: (0, i, 0)),
        pl.BlockSpec((R, D), lambda i: (i, 0)),
        pl.BlockSpec((3, D, D), lambda i: (0, 0, 0)),
        pl.BlockSpec((1, D), lambda i: (0, 0)),
    ],
    out_specs=[pl.BlockSpec((R, D), lambda i: (i, 0))] * 2,
    out_shape=[jax.ShapeDtypeStruct((N, D), jnp.float32)] * 2,
)


def _last_body(h_ref, y1_ref, y2p_ref, nb_ref, w_ref, b_ref,
               wpt_ref, bp_ref, out_ref, acc_ref):
    i = pl.program_id(0)

    @pl.when(i == 0)
    def _():
        acc_ref[...] = jnp.zeros_like(acc_ref)
        out_ref[...] = jnp.zeros((1, 1), jnp.float32)

    y2 = y2p_ref[0] + y2p_ref[1]
    dot = functools.partial(jnp.dot, preferred_element_type=jnp.float32,
                            precision=lax.Precision.HIGHEST)
    part = dot(y1_ref[...], w_ref[1]) + dot(y2, w_ref[2])
    out = dot(h_ref[...], w_ref[0]) + nb_ref[...] * part + b_ref[...]
    hn = jnp.maximum(out, jnp.float32(0.0))
    acc_ref[...] += hn.reshape(R // 8, 8, D).sum(axis=0)

    @pl.when(i == GRID - 1)
    def _():
        tot = acc_ref[...].sum(axis=0, keepdims=True)
        val = jnp.sum(tot * wpt_ref[...]) / N + bp_ref[0, 0]
        out_ref[...] = val.reshape(1, 1)


_last_call = pl.pallas_call(
    _last_body,
    grid=(GRID,),
    in_specs=[
        pl.BlockSpec((R, D), lambda i: (i, 0)),
        pl.BlockSpec((R, D), lambda i: (i, 0)),
        pl.BlockSpec((NC, R, D), lambda i: (0, i, 0)),
        pl.BlockSpec((R, D), lambda i: (i, 0)),
        pl.BlockSpec((3, D, D), lambda i: (0, 0, 0)),
        pl.BlockSpec((1, D), lambda i: (0, 0)),
        pl.BlockSpec((1, D), lambda i: (0, 0)),
        pl.BlockSpec((1, 1), lambda i: (0, 0)),
    ],
    out_specs=pl.BlockSpec((1, 1), lambda i: (0, 0)),
    out_shape=jax.ShapeDtypeStruct((1, 1), jnp.float32),
    scratch_shapes=[pltpu.VMEM((8, D), jnp.float32)],
)


def kernel(x, edge_index, W1, b1, W2, b2, W3, b3, W4, b4, W5, b5, Wp, bp):
    src = edge_index[0]
    dst = edge_index[1]
    sd = jnp.stack([src.reshape(E // C, C), dst.reshape(E // C, C)], axis=1)
    deg_kernel, spmm_kernel = _sc_kernels()

    deg2 = deg_kernel(sd)
    normb, normb2, g, h = _prep_call(deg2, x)

    for W, b in ((W1, b1), (W2, b2), (W3, b3), (W4, b4)):
        y1p = spmm_kernel(g, sd)
        y1, g2 = _scale_call(y1p, normb2)
        y2p = spmm_kernel(g2, sd)
        h, g = _layer_call(h, y1, y2p, normb, W.reshape(3, D, D),
                           b.reshape(1, D))

    y1p = spmm_kernel(g, sd)
    y1, g2 = _scale_call(y1p, normb2)
    y2p = spmm_kernel(g2, sd)
    return _last_call(h, y1, y2p, normb, W5.reshape(3, D, D),
                      b5.reshape(1, D), Wp.reshape(1, D), bp.reshape(1, 1))


# speedup vs baseline: 9.9266x; 1.0041x over previous
import functools

import jax
import jax.numpy as jnp
from jax import lax
from jax.experimental import pallas as pl
from jax.experimental.pallas import tpu as pltpu
from jax.experimental.pallas import tpu_sc as plsc

N = 10000
E = 320000
D = 128
NC = 2
NS = 16
NW = NC * NS
EPW = E // NW
C = 80
NCHUNK = EPW // C
NP = 10240
RPT = NP // NS
ZR = 128
DEGW = 128

def _fill_vmem(ref, rows, width, value):
    def body(i, _):
        for j in range(width // 16):
            ref[i, pl.ds(16 * j, 16)] = jnp.full((16,), value, jnp.float32)
        return 0
    lax.fori_loop(0, rows, body, 0, unroll=False)


@functools.cache
def _sc_kernels():
    mesh = plsc.VectorSubcoreMesh(core_axis_name="c", subcore_axis_name="s",
                                  num_cores=NC, num_subcores=NS)

    @functools.partial(
        pl.kernel,
        mesh=mesh,
        out_type=jax.ShapeDtypeStruct((NC, NP, DEGW), jnp.float32),
        scratch_types=[
            [pltpu.VMEM((2, C), jnp.int32)] * 8,
            pltpu.VMEM((C, DEGW), jnp.float32),
            pltpu.VMEM((C, DEGW), jnp.float32),
            pltpu.VMEM_SHARED((NP, DEGW), jnp.float32),
            [pltpu.SemaphoreType.DMA] * 8,
            [pltpu.SemaphoreType.DMA] * 4,
        ],
    )
    def deg_kernel(sd_hbm, out_hbm, q, ones_v, zbuf_v, acc_sh, isem, ssem):
        cid = lax.axis_index("c")
        sid = lax.axis_index("s")
        wid = cid * NS + sid
        cbase = wid * NCHUNK

        def idx_copy(c, jq):
            pltpu.async_copy(sd_hbm.at[cbase + c], q[jq], isem[jq])

        def idx_wait(jq):
            pltpu.make_async_copy(sd_hbm.at[cbase], q[jq], isem[jq]).wait()

        def scatter(jq, jr):
            pltpu.async_copy(ones_v, acc_sh.at[q[jq].at[1]], ssem[jr],
                             add=True)

        def scatter_wait(jr):
            pltpu.make_async_copy(ones_v, acc_sh.at[q[0].at[1]],
                                  ssem[jr]).wait()

        for c in range(6):
            idx_copy(c, c)
        _fill_vmem(ones_v, C, DEGW, 1.0)
        _fill_vmem(zbuf_v, C, DEGW, 0.0)
        zcopies = [
            pltpu.async_copy(zbuf_v, acc_sh.at[pl.ds(sid * RPT + k * C, C)],
                             ssem[3])
            for k in range(RPT // C)
        ]
        for zc in zcopies:
            zc.wait()
        plsc.subcore_barrier()

        NK = NCHUNK // 8
        def body(k, _):
            for j in range(8):
                c = 8 * k + j
                idx_wait(j % 8)
                scatter(j, j % 4)
                if j < 2:
                    @pl.when(k > 0)
                    def _():
                        scatter_wait((j + 2) % 4)
                else:
                    scatter_wait((j + 2) % 4)
                if j == 7:
                    @pl.when(k < NK - 1)
                    def _():
                        idx_copy(c + 6, (j + 6) % 8)
                else:
                    idx_copy(c + 6, (j + 6) % 8)
            return 0
        lax.fori_loop(0, NK, body, 0, unroll=False)

        idx_wait(0); scatter(0, 0); scatter_wait(2)
        idx_wait(1); scatter(1, 1); scatter_wait(3)
        idx_wait(2); scatter(2, 2); scatter_wait(0)
        idx_wait(3); scatter(3, 3); scatter_wait(1)
        idx_wait(4); scatter(4, 0); scatter_wait(2)
        scatter_wait(3)
        scatter_wait(0)

        plsc.subcore_barrier()
        pltpu.sync_copy(acc_sh.at[pl.ds(sid * RPT, RPT)],
                        out_hbm.at[cid, pl.ds(sid * RPT, RPT)])

    @functools.partial(
        pl.kernel,
        mesh=mesh,
        out_type=jax.ShapeDtypeStruct((NC, NP, D), jnp.float32),
        scratch_types=[
            [pltpu.VMEM((2, C), jnp.int32)] * 8,
            [pltpu.VMEM((C, D), jnp.float32)] * 4,
            pltpu.VMEM_SHARED((NP, D), jnp.float32),
            [pltpu.SemaphoreType.DMA] * 8,
            [pltpu.SemaphoreType.DMA] * 4,
            [pltpu.SemaphoreType.DMA] * 4,
        ],
    )
    def spmm_kernel(g_hbm, sd_hbm, out_hbm,
                    q, rows, y_sh, isem, gsem, ssem):
        cid = lax.axis_index("c")
        sid = lax.axis_index("s")
        wid = cid * NS + sid
        cbase = wid * NCHUNK

        def idx_copy(c, jq):
            pltpu.async_copy(sd_hbm.at[cbase + c], q[jq], isem[jq])

        def idx_wait(jq):
            pltpu.make_async_copy(sd_hbm.at[cbase], q[jq], isem[jq]).wait()

        def gather(jq, jr):
            pltpu.async_copy(g_hbm.at[q[jq].at[0]], rows[jr], gsem[jr])

        def gather_wait(jr):
            pltpu.make_async_copy(g_hbm.at[q[0].at[0]], rows[jr],
                                  gsem[jr]).wait()

        def scatter(jq, jr):
            pltpu.async_copy(rows[jr], y_sh.at[q[jq].at[1]], ssem[jr],
                             add=True)

        def scatter_wait(jr):
            pltpu.make_async_copy(rows[jr], y_sh.at[q[0].at[1]],
                                  ssem[jr]).wait()

        for c in range(6):
            idx_copy(c, c)
        _fill_vmem(rows[0], C, D, 0.0)
        zcopies = [
            pltpu.async_copy(rows[0], y_sh.at[pl.ds(sid * RPT + k * C, C)],
                             gsem[3])
            for k in range(RPT // C)
        ]
        for zc in zcopies:
            zc.wait()
        plsc.subcore_barrier()
        idx_wait(0)
        gather(0, 0)
        idx_wait(1)
        gather(1, 1)

        NK = NCHUNK // 8
        def body(k, _):
            for j in range(8):
                c = 8 * k + j
                gather_wait(j % 4)
                scatter(j, j % 4)
                if j < 2:
                    @pl.when(k > 0)
                    def _():
                        scatter_wait((j + 2) % 4)
                else:
                    scatter_wait((j + 2) % 4)
                if j == 7:
                    @pl.when(k < NK - 1)
                    def _():
                        idx_copy(c + 6, (j + 6) % 8)
                else:
                    idx_copy(c + 6, (j + 6) % 8)
                idx_wait((j + 2) % 8)
                gather((j + 2) % 8, (j + 2) % 4)
            return 0
        lax.fori_loop(0, NK, body, 0, unroll=False)

        b0 = 8 * NK
        gather_wait(0); scatter(0, 0); scatter_wait(2)
        idx_wait(2); gather(2, 2)
        gather_wait(1); scatter(1, 1); scatter_wait(3)
        idx_wait(3); gather(3, 3)
        gather_wait(2); scatter(2, 2); scatter_wait(0)
        idx_wait(4); gather(4, 0)
        gather_wait(3); scatter(3, 3); scatter_wait(1)
        gather_wait(0); scatter(4, 0); scatter_wait(2)
        scatter_wait(3)
        scatter_wait(0)
        del b0

        plsc.subcore_barrier()
        pltpu.sync_copy(y_sh.at[pl.ds(sid * RPT, RPT)],
                        out_hbm.at[cid, pl.ds(sid * RPT, RPT)])

    return deg_kernel, spmm_kernel


R = 400
GRID = N // R
_F32MAX = 3.4028234663852886e38


def _prep_body(deg2_ref, x_ref, normb_ref, normb2_ref, g1_ref, h0_ref):
    deg = deg2_ref[0, :, 0:1] + deg2_ref[1, :, 0:1]
    nrm = lax.rsqrt(jnp.maximum(deg, 1.0))
    nb = jnp.broadcast_to(nrm, (R, D))
    xb = x_ref[...]
    h0 = jnp.where(jnp.isnan(xb), 0.0, xb)
    h0 = jnp.clip(h0, -_F32MAX, _F32MAX)
    normb_ref[...] = nb
    normb2_ref[...] = nb * nb
    g1_ref[...] = nb * h0
    h0_ref[...] = h0


_prep_call = pl.pallas_call(
    _prep_body,
    grid=(GRID,),
    in_specs=[
        pl.BlockSpec((NC, R, DEGW), lambda i: (0, i, 0)),
        pl.BlockSpec((R, D), lambda i: (i, 0)),
    ],
    out_specs=[pl.BlockSpec((R, D), lambda i: (i, 0))] * 4,
    out_shape=[jax.ShapeDtypeStruct((N, D), jnp.float32)] * 4,
)


def _scale_body(ypair_ref, nb2_ref, ysum_ref, g2_ref):
    y = ypair_ref[0] + ypair_ref[1]
    ysum_ref[...] = y
    g2_ref[...] = nb2_ref[...] * y


_scale_call = pl.pallas_call(
    _scale_body,
    grid=(GRID,),
    in_specs=[
        pl.BlockSpec((NC, R, D), lambda i: (0, i, 0)),
        pl.BlockSpec((R, D), lambda i: (i, 0)),
    ],
    out_specs=[pl.BlockSpec((R, D), lambda i: (i, 0))] * 2,
    out_shape=[jax.ShapeDtypeStruct((N, D), jnp.float32)] * 2,
)


def _layer_body(h_ref, y1_ref, y2p_ref, nb_ref, w_ref, b_ref, hn_ref, gn_ref):
    y2 = y2p_ref[0] + y2p_ref[1]
    dot = functools.partial(jnp.dot, preferred_element_type=jnp.float32,
                            precision=lax.Precision.HIGHEST)
    part = dot(y1_ref[...], w_ref[1]) + dot(y2, w_ref[2])
    out = dot(h_ref[...], w_ref[0]) + nb_ref[...] * part + b_ref[...]
    hn = jnp.maximum(out, jnp.float32(0.0))
    hn_ref[...] = hn
    gn_ref[...] = nb_ref[...] * hn


_layer_call = pl.pallas_call(
    _layer_body,
    grid=(GRID,),
    in_specs=[
        pl.BlockSpec((R, D), lambda i: (i, 0)),
        pl.BlockSpec((R, D), lambda i: (i, 0)),
        pl.BlockSpec((NC, R, D), lambda i: (0, i, 0)),
        pl.BlockSpec((R, D), lambda i: (i, 0)),
        pl.BlockSpec((3, D, D), lambda i: (0, 0, 0)),
        pl.BlockSpec((1, D), lambda i: (0, 0)),
    ],
    out_specs=[pl.BlockSpec((R, D), lambda i: (i, 0))] * 2,
    out_shape=[jax.ShapeDtypeStruct((N, D), jnp.float32)] * 2,
)


def _last_body(h_ref, y1_ref, y2p_ref, nb_ref, w_ref, b_ref,
               wpt_ref, bp_ref, out_ref, acc_ref):
    i = pl.program_id(0)

    @pl.when(i == 0)
    def _():
        acc_ref[...] = jnp.zeros_like(acc_ref)
        out_ref[...] = jnp.zeros((1, 1), jnp.float32)

    y2 = y2p_ref[0] + y2p_ref[1]
    dot = functools.partial(jnp.dot, preferred_element_type=jnp.float32,
                            precision=lax.Precision.HIGHEST)
    part = dot(y1_ref[...], w_ref[1]) + dot(y2, w_ref[2])
    out = dot(h_ref[...], w_ref[0]) + nb_ref[...] * part + b_ref[...]
    hn = jnp.maximum(out, jnp.float32(0.0))
    acc_ref[...] += hn.reshape(R // 8, 8, D).sum(axis=0)

    @pl.when(i == GRID - 1)
    def _():
        tot = acc_ref[...].sum(axis=0, keepdims=True)
        val = jnp.sum(tot * wpt_ref[...]) / N + bp_ref[0, 0]
        out_ref[...] = val.reshape(1, 1)


_last_call = pl.pallas_call(
    _last_body,
    grid=(GRID,),
    in_specs=[
        pl.BlockSpec((R, D), lambda i: (i, 0)),
        pl.BlockSpec((R, D), lambda i: (i, 0)),
        pl.BlockSpec((NC, R, D), lambda i: (0, i, 0)),
        pl.BlockSpec((R, D), lambda i: (i, 0)),
        pl.BlockSpec((3, D, D), lambda i: (0, 0, 0)),
        pl.BlockSpec((1, D), lambda i: (0, 0)),
        pl.BlockSpec((1, D), lambda i: (0, 0)),
        pl.BlockSpec((1, 1), lambda i: (0, 0)),
    ],
    out_specs=pl.BlockSpec((1, 1), lambda i: (0, 0)),
    out_shape=jax.ShapeDtypeStruct((1, 1), jnp.float32),
    scratch_shapes=[pltpu.VMEM((8, D), jnp.float32)],
)


def kernel(x, edge_index, W1, b1, W2, b2, W3, b3, W4, b4, W5, b5, Wp, bp):
    src = edge_index[0]
    dst = edge_index[1]
    sd = jnp.stack([src.reshape(E // C, C), dst.reshape(E // C, C)], axis=1)
    deg_kernel, spmm_kernel = _sc_kernels()

    deg2 = deg_kernel(sd)
    normb, normb2, g, h = _prep_call(deg2, x)

    for W, b in ((W1, b1), (W2, b2), (W3, b3), (W4, b4)):
        y1p = spmm_kernel(g, sd)
        y1, g2 = _scale_call(y1p, normb2)
        y2p = spmm_kernel(g2, sd)
        h, g = _layer_call(h, y1, y2p, normb, W.reshape(3, D, D),
                           b.reshape(1, D))

    y1p = spmm_kernel(g, sd)
    y1, g2 = _scale_call(y1p, normb2)
    y2p = spmm_kernel(g2, sd)
    return _last_call(h, y1, y2p, normb, W5.reshape(3, D, D),
                      b5.reshape(1, D), Wp.reshape(1, D), bp.reshape(1, 1))


# speedup vs baseline: 10.6589x; 1.0738x over previous
import functools

import jax
import jax.numpy as jnp
from jax import lax
from jax.experimental import pallas as pl
from jax.experimental.pallas import tpu as pltpu
from jax.experimental.pallas import tpu_sc as plsc

N = 10000
E = 320000
D = 128
NC = 2
NS = 16
NW = NC * NS
EPW = E // NW
C = 80
NCHUNK = EPW // C
NP = 10240
RPT = NP // NS
ZR = 128
DEGW = 128

def _fill_vmem(ref, rows, width, value):
    def body(i, _):
        for j in range(width // 16):
            ref[i, pl.ds(16 * j, 16)] = jnp.full((16,), value, jnp.float32)
        return 0
    lax.fori_loop(0, rows, body, 0, unroll=False)


@functools.cache
def _sc_kernels():
    mesh = plsc.VectorSubcoreMesh(core_axis_name="c", subcore_axis_name="s",
                                  num_cores=NC, num_subcores=NS)

    @functools.partial(
        pl.kernel,
        mesh=mesh,
        out_type=jax.ShapeDtypeStruct((NC, NP, DEGW), jnp.float32),
        scratch_types=[
            [pltpu.VMEM((2, C), jnp.int32)] * 8,
            pltpu.VMEM((C, DEGW), jnp.float32),
            pltpu.VMEM((C, DEGW), jnp.float32),
            pltpu.VMEM_SHARED((NP, DEGW), jnp.float32),
            [pltpu.SemaphoreType.DMA] * 8,
            [pltpu.SemaphoreType.DMA] * 4,
        ],
    )
    def deg_kernel(sd_hbm, out_hbm, q, ones_v, zbuf_v, acc_sh, isem, ssem):
        cid = lax.axis_index("c")
        sid = lax.axis_index("s")
        wid = cid * NS + sid
        cbase = wid * NCHUNK

        def idx_copy(c, jq):
            pltpu.async_copy(sd_hbm.at[cbase + c], q[jq], isem[jq])

        def idx_wait(jq):
            pltpu.make_async_copy(sd_hbm.at[cbase], q[jq], isem[jq]).wait()

        def scatter(jq, jr):
            pltpu.async_copy(ones_v, acc_sh.at[q[jq].at[1]], ssem[jr],
                             add=True)

        def scatter_wait(jr):
            pltpu.make_async_copy(ones_v, acc_sh.at[q[0].at[1]],
                                  ssem[jr]).wait()

        for c in range(6):
            idx_copy(c, c)
        _fill_vmem(ones_v, C, DEGW, 1.0)
        _fill_vmem(zbuf_v, C, DEGW, 0.0)
        zcopies = [
            pltpu.async_copy(zbuf_v, acc_sh.at[pl.ds(sid * RPT + k * C, C)],
                             ssem[3])
            for k in range(RPT // C)
        ]
        for zc in zcopies:
            zc.wait()
        plsc.subcore_barrier()

        NK = NCHUNK // 8
        def body(k, _):
            for j in range(8):
                c = 8 * k + j
                idx_wait(j % 8)
                scatter(j, j % 4)
                if j < 2:
                    @pl.when(k > 0)
                    def _():
                        scatter_wait((j + 2) % 4)
                else:
                    scatter_wait((j + 2) % 4)
                if j == 7:
                    @pl.when(k < NK - 1)
                    def _():
                        idx_copy(c + 6, (j + 6) % 8)
                else:
                    idx_copy(c + 6, (j + 6) % 8)
            return 0
        lax.fori_loop(0, NK, body, 0, unroll=False)

        idx_wait(0); scatter(0, 0); scatter_wait(2)
        idx_wait(1); scatter(1, 1); scatter_wait(3)
        idx_wait(2); scatter(2, 2); scatter_wait(0)
        idx_wait(3); scatter(3, 3); scatter_wait(1)
        idx_wait(4); scatter(4, 0); scatter_wait(2)
        scatter_wait(3)
        scatter_wait(0)

        plsc.subcore_barrier()
        pltpu.sync_copy(acc_sh.at[pl.ds(sid * RPT, RPT)],
                        out_hbm.at[cid, pl.ds(sid * RPT, RPT)])

    @functools.partial(
        pl.kernel,
        mesh=mesh,
        out_type=jax.ShapeDtypeStruct((NC, NP, D), jnp.float32),
        scratch_types=[
            [pltpu.VMEM((2, C), jnp.int32)] * 8,
            [pltpu.VMEM((C, D), jnp.float32)] * 4,
            pltpu.VMEM_SHARED((NP, D), jnp.float32),
            [pltpu.SemaphoreType.DMA] * 8,
            [pltpu.SemaphoreType.DMA] * 4,
            [pltpu.SemaphoreType.DMA] * 4,
        ],
    )
    def spmm_kernel(g_hbm, sd_hbm, out_hbm,
                    q, rows, y_sh, isem, gsem, ssem):
        cid = lax.axis_index("c")
        sid = lax.axis_index("s")
        wid = cid * NS + sid
        cbase = wid * NCHUNK

        def idx_copy(c, jq):
            pltpu.async_copy(sd_hbm.at[cbase + c], q[jq], isem[jq])

        def idx_wait(jq):
            pltpu.make_async_copy(sd_hbm.at[cbase], q[jq], isem[jq]).wait()

        def gather(jq, jr):
            pltpu.async_copy(g_hbm.at[q[jq].at[0]], rows[jr], gsem[jr])

        def gather_wait(jr):
            pltpu.make_async_copy(g_hbm.at[q[0].at[0]], rows[jr],
                                  gsem[jr]).wait()

        def scatter(jq, jr):
            pltpu.async_copy(rows[jr], y_sh.at[q[jq].at[1]], ssem[jr],
                             add=True)

        def scatter_wait(jr):
            pltpu.make_async_copy(rows[jr], y_sh.at[q[0].at[1]],
                                  ssem[jr]).wait()

        for c in range(6):
            idx_copy(c, c)
        _fill_vmem(rows[0], C, D, 0.0)
        zcopies = [
            pltpu.async_copy(rows[0], y_sh.at[pl.ds(sid * RPT + k * C, C)],
                             gsem[3])
            for k in range(RPT // C)
        ]
        for zc in zcopies:
            zc.wait()
        plsc.subcore_barrier()
        idx_wait(0)
        gather(0, 0)
        idx_wait(1)
        gather(1, 1)

        NK = NCHUNK // 8
        def body(k, _):
            for j in range(8):
                c = 8 * k + j
                gather_wait(j % 4)
                scatter(j, j % 4)
                if j < 2:
                    @pl.when(k > 0)
                    def _():
                        scatter_wait((j + 2) % 4)
                else:
                    scatter_wait((j + 2) % 4)
                if j == 7:
                    @pl.when(k < NK - 1)
                    def _():
                        idx_copy(c + 6, (j + 6) % 8)
                else:
                    idx_copy(c + 6, (j + 6) % 8)
                idx_wait((j + 2) % 8)
                gather((j + 2) % 8, (j + 2) % 4)
            return 0
        lax.fori_loop(0, NK, body, 0, unroll=False)

        gather_wait(0); scatter(0, 0); scatter_wait(2)
        idx_wait(2); gather(2, 2)
        gather_wait(1); scatter(1, 1); scatter_wait(3)
        idx_wait(3); gather(3, 3)
        gather_wait(2); scatter(2, 2); scatter_wait(0)
        idx_wait(4); gather(4, 0)
        gather_wait(3); scatter(3, 3); scatter_wait(1)
        gather_wait(0); scatter(4, 0); scatter_wait(2)
        scatter_wait(3)
        scatter_wait(0)

        plsc.subcore_barrier()
        pltpu.sync_copy(y_sh.at[pl.ds(sid * RPT, RPT)],
                        out_hbm.at[cid, pl.ds(sid * RPT, RPT)])

    return deg_kernel, spmm_kernel


R = 2000
GRID = N // R
_F32MAX = 3.4028234663852886e38


def _prep_body(deg2_ref, x_ref, normb_ref, normb2_ref, g1_ref, h0_ref):
    deg = deg2_ref[0, :, 0:1] + deg2_ref[1, :, 0:1]
    nrm = lax.rsqrt(jnp.maximum(deg, 1.0))
    nb = jnp.broadcast_to(nrm, (R, D))
    xb = x_ref[...]
    h0 = jnp.where(jnp.isnan(xb), 0.0, xb)
    h0 = jnp.clip(h0, -_F32MAX, _F32MAX)
    normb_ref[...] = nb
    normb2_ref[...] = nb * nb
    g1_ref[...] = nb * h0
    h0_ref[...] = h0


_prep_call = pl.pallas_call(
    _prep_body,
    grid=(GRID,),
    in_specs=[
        pl.BlockSpec((NC, R, DEGW), lambda i: (0, i, 0)),
        pl.BlockSpec((R, D), lambda i: (i, 0)),
    ],
    out_specs=[pl.BlockSpec((R, D), lambda i: (i, 0))] * 4,
    out_shape=[jax.ShapeDtypeStruct((N, D), jnp.float32)] * 4,
)


def _scale_body(ypair_ref, nb2_ref, ysum_ref, g2_ref):
    y = ypair_ref[0] + ypair_ref[1]
    ysum_ref[...] = y
    g2_ref[...] = nb2_ref[...] * y


_scale_call = pl.pallas_call(
    _scale_body,
    grid=(GRID,),
    in_specs=[
        pl.BlockSpec((NC, R, D), lambda i: (0, i, 0)),
        pl.BlockSpec((R, D), lambda i: (i, 0)),
    ],
    out_specs=[pl.BlockSpec((R, D), lambda i: (i, 0))] * 2,
    out_shape=[jax.ShapeDtypeStruct((N, D), jnp.float32)] * 2,
)


def _layer_body(h_ref, y1_ref, y2p_ref, nb_ref, w_ref, b_ref, hn_ref, gn_ref):
    y2 = y2p_ref[0] + y2p_ref[1]
    dot = functools.partial(jnp.dot, preferred_element_type=jnp.float32,
                            precision=lax.Precision.HIGHEST)
    part = dot(y1_ref[...], w_ref[1]) + dot(y2, w_ref[2])
    out = dot(h_ref[...], w_ref[0]) + nb_ref[...] * part + b_ref[...]
    hn = jnp.maximum(out, jnp.float32(0.0))
    hn_ref[...] = hn
    gn_ref[...] = nb_ref[...] * hn


_layer_call = pl.pallas_call(
    _layer_body,
    grid=(GRID,),
    in_specs=[
        pl.BlockSpec((R, D), lambda i: (i, 0)),
        pl.BlockSpec((R, D), lambda i: (i, 0)),
        pl.BlockSpec((NC, R, D), lambda i: (0, i, 0)),
        pl.BlockSpec((R, D), lambda i: (i, 0)),
        pl.BlockSpec((3, D, D), lambda i: (0, 0, 0)),
        pl.BlockSpec((1, D), lambda i: (0, 0)),
    ],
    out_specs=[pl.BlockSpec((R, D), lambda i: (i, 0))] * 2,
    out_shape=[jax.ShapeDtypeStruct((N, D), jnp.float32)] * 2,
)


def _last_body(h_ref, y1_ref, y2p_ref, nb_ref, w_ref, b_ref,
               wpt_ref, bp_ref, out_ref, acc_ref):
    i = pl.program_id(0)

    @pl.when(i == 0)
    def _():
        acc_ref[...] = jnp.zeros_like(acc_ref)
        out_ref[...] = jnp.zeros((1, 1), jnp.float32)

    y2 = y2p_ref[0] + y2p_ref[1]
    dot = functools.partial(jnp.dot, preferred_element_type=jnp.float32,
                            precision=lax.Precision.HIGHEST)
    part = dot(y1_ref[...], w_ref[1]) + dot(y2, w_ref[2])
    out = dot(h_ref[...], w_ref[0]) + nb_ref[...] * part + b_ref[...]
    hn = jnp.maximum(out, jnp.float32(0.0))
    acc_ref[...] += hn.reshape(R // 8, 8, D).sum(axis=0)

    @pl.when(i == GRID - 1)
    def _():
        tot = acc_ref[...].sum(axis=0, keepdims=True)
        val = jnp.sum(tot * wpt_ref[...]) / N + bp_ref[0, 0]
        out_ref[...] = val.reshape(1, 1)


_last_call = pl.pallas_call(
    _last_body,
    grid=(GRID,),
    in_specs=[
        pl.BlockSpec((R, D), lambda i: (i, 0)),
        pl.BlockSpec((R, D), lambda i: (i, 0)),
        pl.BlockSpec((NC, R, D), lambda i: (0, i, 0)),
        pl.BlockSpec((R, D), lambda i: (i, 0)),
        pl.BlockSpec((3, D, D), lambda i: (0, 0, 0)),
        pl.BlockSpec((1, D), lambda i: (0, 0)),
        pl.BlockSpec((1, D), lambda i: (0, 0)),
        pl.BlockSpec((1, 1), lambda i: (0, 0)),
    ],
    out_specs=pl.BlockSpec((1, 1), lambda i: (0, 0)),
    out_shape=jax.ShapeDtypeStruct((1, 1), jnp.float32),
    scratch_shapes=[pltpu.VMEM((8, D), jnp.float32)],
)


def kernel(x, edge_index, W1, b1, W2, b2, W3, b3, W4, b4, W5, b5, Wp, bp):
    src = edge_index[0]
    dst = edge_index[1]
    sd = jnp.stack([src.reshape(E // C, C), dst.reshape(E // C, C)], axis=1)
    deg_kernel, spmm_kernel = _sc_kernels()

    deg2 = deg_kernel(sd)
    normb, normb2, g, h = _prep_call(deg2, x)

    for W, b in ((W1, b1), (W2, b2), (W3, b3), (W4, b4)):
        y1p = spmm_kernel(g, sd)
        y1, g2 = _scale_call(y1p, normb2)
        y2p = spmm_kernel(g2, sd)
        h, g = _layer_call(h, y1, y2p, normb, W.reshape(3, D, D),
                           b.reshape(1, D))

    y1p = spmm_kernel(g, sd)
    y1, g2 = _scale_call(y1p, normb2)
    y2p = spmm_kernel(g2, sd)
    return _last_call(h, y1, y2p, normb, W5.reshape(3, D, D),
                      b5.reshape(1, D), Wp.reshape(1, D), bp.reshape(1, 1))


# speedup vs baseline: 10.6763x; 1.0016x over previous
import functools

import jax
import jax.numpy as jnp
from jax import lax
from jax.experimental import pallas as pl
from jax.experimental.pallas import tpu as pltpu
from jax.experimental.pallas import tpu_sc as plsc

N = 10000
E = 320000
D = 128
NC = 2
NS = 16
NW = NC * NS
EPW = E // NW
C = 80
NCHUNK = EPW // C
NP = 10240
RPT = NP // NS
ZR = 128
DEGW = 128

def _fill_vmem(ref, rows, width, value):
    def body(i, _):
        for j in range(width // 16):
            ref[i, pl.ds(16 * j, 16)] = jnp.full((16,), value, jnp.float32)
        return 0
    lax.fori_loop(0, rows, body, 0, unroll=False)


@functools.cache
def _sc_kernels():
    mesh = plsc.VectorSubcoreMesh(core_axis_name="c", subcore_axis_name="s",
                                  num_cores=NC, num_subcores=NS)

    @functools.partial(
        pl.kernel,
        mesh=mesh,
        out_type=jax.ShapeDtypeStruct((NC, NP, DEGW), jnp.float32),
        scratch_types=[
            [pltpu.VMEM((2, C), jnp.int32)] * 8,
            pltpu.VMEM((C, DEGW), jnp.float32),
            pltpu.VMEM((C, DEGW), jnp.float32),
            pltpu.VMEM_SHARED((NP, DEGW), jnp.float32),
            [pltpu.SemaphoreType.DMA] * 8,
            [pltpu.SemaphoreType.DMA] * 4,
        ],
    )
    def deg_kernel(sd_hbm, out_hbm, q, ones_v, zbuf_v, acc_sh, isem, ssem):
        cid = lax.axis_index("c")
        sid = lax.axis_index("s")
        wid = cid * NS + sid
        cbase = wid * NCHUNK

        def idx_copy(c, jq):
            pltpu.async_copy(sd_hbm.at[cbase + c], q[jq], isem[jq])

        def idx_wait(jq):
            pltpu.make_async_copy(sd_hbm.at[cbase], q[jq], isem[jq]).wait()

        def scatter(jq, jr):
            pltpu.async_copy(ones_v, acc_sh.at[q[jq].at[1]], ssem[jr],
                             add=True)

        def scatter_wait(jr):
            pltpu.make_async_copy(ones_v, acc_sh.at[q[0].at[1]],
                                  ssem[jr]).wait()

        for c in range(6):
            idx_copy(c, c)
        _fill_vmem(ones_v, C, DEGW, 1.0)
        _fill_vmem(zbuf_v, C, DEGW, 0.0)
        zcopies = [
            pltpu.async_copy(zbuf_v, acc_sh.at[pl.ds(sid * RPT + k * C, C)],
                             ssem[3])
            for k in range(RPT // C)
        ]
        for zc in zcopies:
            zc.wait()
        plsc.subcore_barrier()

        NK = NCHUNK // 8
        def body(k, _):
            for j in range(8):
                c = 8 * k + j
                idx_wait(j % 8)
                scatter(j, j % 4)
                if j < 2:
                    @pl.when(k > 0)
                    def _():
                        scatter_wait((j + 2) % 4)
                else:
                    scatter_wait((j + 2) % 4)
                if j == 7:
                    @pl.when(k < NK - 1)
                    def _():
                        idx_copy(c + 6, (j + 6) % 8)
                else:
                    idx_copy(c + 6, (j + 6) % 8)
            return 0
        lax.fori_loop(0, NK, body, 0, unroll=False)

        idx_wait(0); scatter(0, 0); scatter_wait(2)
        idx_wait(1); scatter(1, 1); scatter_wait(3)
        idx_wait(2); scatter(2, 2); scatter_wait(0)
        idx_wait(3); scatter(3, 3); scatter_wait(1)
        idx_wait(4); scatter(4, 0); scatter_wait(2)
        scatter_wait(3)
        scatter_wait(0)

        plsc.subcore_barrier()
        pltpu.sync_copy(acc_sh.at[pl.ds(sid * RPT, RPT)],
                        out_hbm.at[cid, pl.ds(sid * RPT, RPT)])

    @functools.partial(
        pl.kernel,
        mesh=mesh,
        out_type=jax.ShapeDtypeStruct((NC, NP, D), jnp.float32),
        scratch_types=[
            [pltpu.VMEM((2, C), jnp.int32)] * 8,
            [pltpu.VMEM((C, D), jnp.float32)] * 4,
            pltpu.VMEM_SHARED((NP, D), jnp.float32),
            [pltpu.SemaphoreType.DMA] * 8,
            [pltpu.SemaphoreType.DMA] * 4,
            [pltpu.SemaphoreType.DMA] * 4,
        ],
    )
    def spmm_kernel(g_hbm, sd_hbm, out_hbm,
                    q, rows, y_sh, isem, gsem, ssem):
        cid = lax.axis_index("c")
        sid = lax.axis_index("s")
        wid = cid * NS + sid
        cbase = wid * NCHUNK

        def idx_copy(c, jq):
            pltpu.async_copy(sd_hbm.at[cbase + c], q[jq], isem[jq])

        def idx_wait(jq):
            pltpu.make_async_copy(sd_hbm.at[cbase], q[jq], isem[jq]).wait()

        def gather(jq, jr):
            pltpu.async_copy(g_hbm.at[q[jq].at[0]], rows[jr], gsem[jr])

        def gather_wait(jr):
            pltpu.make_async_copy(g_hbm.at[q[0].at[0]], rows[jr],
                                  gsem[jr]).wait()

        def scatter(jq, jr):
            pltpu.async_copy(rows[jr], y_sh.at[q[jq].at[1]], ssem[jr],
                             add=True)

        def scatter_wait(jr):
            pltpu.make_async_copy(rows[jr], y_sh.at[q[0].at[1]],
                                  ssem[jr]).wait()

        for c in range(6):
            idx_copy(c, c)
        _fill_vmem(rows[0], C, D, 0.0)
        zcopies = [
            pltpu.async_copy(rows[0], y_sh.at[pl.ds(sid * RPT + k * C, C)],
                             gsem[3])
            for k in range(RPT // C)
        ]
        for zc in zcopies:
            zc.wait()
        plsc.subcore_barrier()
        idx_wait(0)
        gather(0, 0)
        idx_wait(1)
        gather(1, 1)

        NK = NCHUNK // 8
        def body(k, _):
            for j in range(8):
                c = 8 * k + j
                gather_wait(j % 4)
                scatter(j, j % 4)
                if j < 2:
                    @pl.when(k > 0)
                    def _():
                        scatter_wait((j + 2) % 4)
                else:
                    scatter_wait((j + 2) % 4)
                if j == 7:
                    @pl.when(k < NK - 1)
                    def _():
                        idx_copy(c + 6, (j + 6) % 8)
                else:
                    idx_copy(c + 6, (j + 6) % 8)
                idx_wait((j + 2) % 8)
                gather((j + 2) % 8, (j + 2) % 4)
            return 0
        lax.fori_loop(0, NK, body, 0, unroll=False)

        gather_wait(0); scatter(0, 0); scatter_wait(2)
        idx_wait(2); gather(2, 2)
        gather_wait(1); scatter(1, 1); scatter_wait(3)
        idx_wait(3); gather(3, 3)
        gather_wait(2); scatter(2, 2); scatter_wait(0)
        idx_wait(4); gather(4, 0)
        gather_wait(3); scatter(3, 3); scatter_wait(1)
        gather_wait(0); scatter(4, 0); scatter_wait(2)
        scatter_wait(3)
        scatter_wait(0)

        plsc.subcore_barrier()
        pltpu.sync_copy(y_sh.at[pl.ds(sid * RPT, RPT)],
                        out_hbm.at[cid, pl.ds(sid * RPT, RPT)])

    return deg_kernel, spmm_kernel


R = 2000
GRID = N // R
_F32MAX = 3.4028234663852886e38


def _prep_body(deg2_ref, x_ref, normb_ref, g1_ref, h0_ref):
    deg = deg2_ref[0, :, 0:1] + deg2_ref[1, :, 0:1]
    nrm = lax.pow(jnp.clip(deg, 1.0, None), -0.5)
    nb = jnp.broadcast_to(nrm, (R, D))
    xb = x_ref[...]
    h0 = jnp.where(jnp.isnan(xb), 0.0, xb)
    h0 = jnp.clip(h0, -_F32MAX, _F32MAX)
    normb_ref[...] = nb
    g1_ref[...] = nb * h0
    h0_ref[...] = h0


_prep_call = pl.pallas_call(
    _prep_body,
    grid=(GRID,),
    in_specs=[
        pl.BlockSpec((NC, R, DEGW), lambda i: (0, i, 0)),
        pl.BlockSpec((R, D), lambda i: (i, 0)),
    ],
    out_specs=[pl.BlockSpec((R, D), lambda i: (i, 0))] * 3,
    out_shape=[jax.ShapeDtypeStruct((N, D), jnp.float32)] * 3,
)


def _scale_body(ypair_ref, nb_ref, m1_ref, g2_ref):
    m1 = (ypair_ref[0] + ypair_ref[1]) * nb_ref[...]
    m1_ref[...] = m1
    g2_ref[...] = m1 * nb_ref[...]


_scale_call = pl.pallas_call(
    _scale_body,
    grid=(GRID,),
    in_specs=[
        pl.BlockSpec((NC, R, D), lambda i: (0, i, 0)),
        pl.BlockSpec((R, D), lambda i: (i, 0)),
    ],
    out_specs=[pl.BlockSpec((R, D), lambda i: (i, 0))] * 2,
    out_shape=[jax.ShapeDtypeStruct((N, D), jnp.float32)] * 2,
)


def _layer_body(h_ref, m1_ref, y2p_ref, nb_ref, w_ref, b_ref, hn_ref, gn_ref):
    nb = nb_ref[...]
    m2 = (y2p_ref[0] + y2p_ref[1]) * nb
    dot = functools.partial(jnp.dot, preferred_element_type=jnp.float32,
                            precision=lax.Precision.HIGHEST)
    out = (dot(h_ref[...], w_ref[0]) + dot(m1_ref[...], w_ref[1])
           + dot(m2, w_ref[2]) + b_ref[...])
    hn = jnp.maximum(out, jnp.float32(0.0))
    hn_ref[...] = hn
    gn_ref[...] = nb * hn


_layer_call = pl.pallas_call(
    _layer_body,
    grid=(GRID,),
    in_specs=[
        pl.BlockSpec((R, D), lambda i: (i, 0)),
        pl.BlockSpec((R, D), lambda i: (i, 0)),
        pl.BlockSpec((NC, R, D), lambda i: (0, i, 0)),
        pl.BlockSpec((R, D), lambda i: (i, 0)),
        pl.BlockSpec((3, D, D), lambda i: (0, 0, 0)),
        pl.BlockSpec((1, D), lambda i: (0, 0)),
    ],
    out_specs=[pl.BlockSpec((R, D), lambda i: (i, 0))] * 2,
    out_shape=[jax.ShapeDtypeStruct((N, D), jnp.float32)] * 2,
)


def _last_body(h_ref, m1_ref, y2p_ref, nb_ref, w_ref, b_ref,
               wpt_ref, bp_ref, out_ref, acc_ref):
    i = pl.program_id(0)

    @pl.when(i == 0)
    def _():
        acc_ref[...] = jnp.zeros_like(acc_ref)
        out_ref[...] = jnp.zeros((1, 1), jnp.float32)

    nb = nb_ref[...]
    m2 = (y2p_ref[0] + y2p_ref[1]) * nb
    dot = functools.partial(jnp.dot, preferred_element_type=jnp.float32,
                            precision=lax.Precision.HIGHEST)
    out = (dot(h_ref[...], w_ref[0]) + dot(m1_ref[...], w_ref[1])
           + dot(m2, w_ref[2]) + b_ref[...])
    hn = jnp.maximum(out, jnp.float32(0.0))
    acc_ref[...] += hn.reshape(R // 8, 8, D).sum(axis=0)

    @pl.when(i == GRID - 1)
    def _():
        tot = acc_ref[...].sum(axis=0, keepdims=True)
        val = jnp.sum(tot * wpt_ref[...]) / N + bp_ref[0, 0]
        out_ref[...] = val.reshape(1, 1)


_last_call = pl.pallas_call(
    _last_body,
    grid=(GRID,),
    in_specs=[
        pl.BlockSpec((R, D), lambda i: (i, 0)),
        pl.BlockSpec((R, D), lambda i: (i, 0)),
        pl.BlockSpec((NC, R, D), lambda i: (0, i, 0)),
        pl.BlockSpec((R, D), lambda i: (i, 0)),
        pl.BlockSpec((3, D, D), lambda i: (0, 0, 0)),
        pl.BlockSpec((1, D), lambda i: (0, 0)),
        pl.BlockSpec((1, D), lambda i: (0, 0)),
        pl.BlockSpec((1, 1), lambda i: (0, 0)),
    ],
    out_specs=pl.BlockSpec((1, 1), lambda i: (0, 0)),
    out_shape=jax.ShapeDtypeStruct((1, 1), jnp.float32),
    scratch_shapes=[pltpu.VMEM((8, D), jnp.float32)],
)


def kernel(x, edge_index, W1, b1, W2, b2, W3, b3, W4, b4, W5, b5, Wp, bp):
    src = edge_index[0]
    dst = edge_index[1]
    sd = jnp.stack([src.reshape(E // C, C), dst.reshape(E // C, C)], axis=1)
    deg_kernel, spmm_kernel = _sc_kernels()

    deg2 = deg_kernel(sd)
    normb, g, h = _prep_call(deg2, x)

    for W, b in ((W1, b1), (W2, b2), (W3, b3), (W4, b4)):
        y1p = spmm_kernel(g, sd)
        m1, g2 = _scale_call(y1p, normb)
        y2p = spmm_kernel(g2, sd)
        h, g = _layer_call(h, m1, y2p, normb, W.reshape(3, D, D),
                           b.reshape(1, D))

    y1p = spmm_kernel(g, sd)
    m1, g2 = _scale_call(y1p, normb)
    y2p = spmm_kernel(g2, sd)
    return _last_call(h, m1, y2p, normb, W5.reshape(3, D, D),
                      b5.reshape(1, D), Wp.reshape(1, D), bp.reshape(1, 1))
